# baseline (speedup 1.0000x reference)
import functools

import jax
import jax.numpy as jnp
from jax.experimental import pallas as pl

N_NODES = 10000
N_EDGES = 160000
D_EDGE = 16
HID = 256
GRAPH_FEATS = 512
MID = 256
TGT = 10
N_LAYERS = 5
BN_EPS = 1e-5

NP = 10240
NT = NP // 256
ET = N_EDGES // 256


def _edge_kernel(ef_ref, w_ref, b_ref, out_ref):
    e = jnp.dot(ef_ref[...], w_ref[0], preferred_element_type=jnp.float32)
    out_ref[0] = e + b_ref[0, 0:1, :]


def _edge_embeddings(edge_feats, eW, eb):
    return pl.pallas_call(
        _edge_kernel,
        grid=(N_LAYERS, ET),
        in_specs=[
            pl.BlockSpec((256, D_EDGE), lambda l, i: (i, 0)),
            pl.BlockSpec((1, D_EDGE, HID), lambda l, i: (l, 0, 0)),
            pl.BlockSpec((1, 8, HID), lambda l, i: (l, 0, 0)),
        ],
        out_specs=pl.BlockSpec((1, 256, HID), lambda l, i: (l, i, 0)),
        out_shape=jax.ShapeDtypeStruct((N_LAYERS, N_EDGES, HID), jnp.float32),
    )(edge_feats, eW, eb)


def _node_kernel(h_ref, agg_ref, vn_ref, eps_ref, w1_ref, b1_ref, w2_ref,
                 b2_ref, hout_ref, hsum_ref, *, inner_relu):
    i = pl.program_id(0)
    hi = h_ref[...] + vn_ref[0:1, :]
    s = eps_ref[0, 0]
    z = s * hi + agg_ref[...]
    z = jnp.maximum(jnp.dot(z, w1_ref[...], preferred_element_type=jnp.float32)
                    + b1_ref[0:1, :], 0.0)
    z = jnp.dot(z, w2_ref[...], preferred_element_type=jnp.float32) + b2_ref[0:1, :]
    if inner_relu:
        z = jnp.maximum(z, 0.0)
    ho = z + hi
    row = i * 256 + jax.lax.broadcasted_iota(jnp.int32, (256, HID), 0)
    valid = row < N_NODES
    hout_ref[...] = jnp.where(valid, ho, 0.0)
    him = jnp.where(valid, hi, 0.0)
    colsum = jnp.sum(him, axis=0, keepdims=True)

    @pl.when(i == 0)
    def _():
        hsum_ref[...] = jnp.zeros_like(hsum_ref)

    hsum_ref[0:1, :] = hsum_ref[0:1, :] + colsum


def _node_block(h, agg, vn8, eps8, w1, b1, w2, b2, inner_relu):
    return pl.pallas_call(
        functools.partial(_node_kernel, inner_relu=inner_relu),
        grid=(NT,),
        in_specs=[
            pl.BlockSpec((256, HID), lambda i: (i, 0)),
            pl.BlockSpec((256, HID), lambda i: (i, 0)),
            pl.BlockSpec((8, HID), lambda i: (0, 0)),
            pl.BlockSpec((8, 128), lambda i: (0, 0)),
            pl.BlockSpec((HID, 2 * HID), lambda i: (0, 0)),
            pl.BlockSpec((8, 2 * HID), lambda i: (0, 0)),
            pl.BlockSpec((2 * HID, HID), lambda i: (0, 0)),
            pl.BlockSpec((8, HID), lambda i: (0, 0)),
        ],
        out_specs=[
            pl.BlockSpec((256, HID), lambda i: (i, 0)),
            pl.BlockSpec((8, HID), lambda i: (0, 0)),
        ],
        out_shape=[
            jax.ShapeDtypeStruct((NP, HID), jnp.float32),
            jax.ShapeDtypeStruct((8, HID), jnp.float32),
        ],
    )(h, agg, vn8, eps8, w1, b1, w2, b2)


def _final_kernel(h_ref, agg_ref, vn_ref, eps_ref, w1_ref, b1_ref, w2_ref,
                  b2_ref, riw_ref, rib_ref, row_ref, rob_ref, g_ref):
    i = pl.program_id(0)
    hi = h_ref[...] + vn_ref[0:1, :]
    s = eps_ref[0, 0]
    z = s * hi + agg_ref[...]
    z = jnp.maximum(jnp.dot(z, w1_ref[...], preferred_element_type=jnp.float32)
                    + b1_ref[0:1, :], 0.0)
    z = jnp.dot(z, w2_ref[...], preferred_element_type=jnp.float32) + b2_ref[0:1, :]
    ho = z + hi
    r = jax.nn.sigmoid(jnp.dot(ho, riw_ref[...], preferred_element_type=jnp.float32)
                       + rib_ref[0:1, :])
    r = jnp.dot(r, row_ref[...], preferred_element_type=jnp.float32) + rob_ref[0:1, :]
    rowid = i * 256 + jax.lax.broadcasted_iota(jnp.int32, (256, GRAPH_FEATS), 0)
    r = jnp.where(rowid < N_NODES, r, 0.0)
    colsum = jnp.sum(r, axis=0, keepdims=True)

    @pl.when(i == 0)
    def _():
        g_ref[...] = jnp.zeros_like(g_ref)

    g_ref[0:1, :] = g_ref[0:1, :] + colsum


def _final_block(h, agg, vn8, eps8, w1, b1, w2, b2, riw, rib, row_, rob):
    return pl.pallas_call(
        _final_kernel,
        grid=(NT,),
        in_specs=[
            pl.BlockSpec((256, HID), lambda i: (i, 0)),
            pl.BlockSpec((256, HID), lambda i: (i, 0)),
            pl.BlockSpec((8, HID), lambda i: (0, 0)),
            pl.BlockSpec((8, 128), lambda i: (0, 0)),
            pl.BlockSpec((HID, 2 * HID), lambda i: (0, 0)),
            pl.BlockSpec((8, 2 * HID), lambda i: (0, 0)),
            pl.BlockSpec((2 * HID, HID), lambda i: (0, 0)),
            pl.BlockSpec((8, HID), lambda i: (0, 0)),
            pl.BlockSpec((HID, GRAPH_FEATS), lambda i: (0, 0)),
            pl.BlockSpec((8, GRAPH_FEATS), lambda i: (0, 0)),
            pl.BlockSpec((GRAPH_FEATS, GRAPH_FEATS), lambda i: (0, 0)),
            pl.BlockSpec((8, GRAPH_FEATS), lambda i: (0, 0)),
        ],
        out_specs=pl.BlockSpec((8, GRAPH_FEATS), lambda i: (0, 0)),
        out_shape=jax.ShapeDtypeStruct((8, GRAPH_FEATS), jnp.float32),
    )(h, agg, vn8, eps8, w1, b1, w2, b2, riw, rib, row_, rob)


def _vn_kernel(vn_ref, hs_ref, w1_ref, b1_ref, w2_ref, b2_ref, out_ref):
    vt = hs_ref[0:1, :] + vn_ref[0:1, :]
    a = jnp.maximum(jnp.dot(vt, w1_ref[...], preferred_element_type=jnp.float32)
                    + b1_ref[0:1, :], 0.0)
    b = jnp.dot(a, w2_ref[...], preferred_element_type=jnp.float32) + b2_ref[0:1, :]
    vnn = vn_ref[0:1, :] + jnp.maximum(b, 0.0)
    out_ref[...] = jnp.broadcast_to(vnn, (8, HID))


def _vn_block(vn8, hsum, w1, b1, w2, b2):
    return pl.pallas_call(
        _vn_kernel,
        in_specs=[pl.BlockSpec((8, HID), lambda: (0, 0)),
                  pl.BlockSpec((8, HID), lambda: (0, 0)),
                  pl.BlockSpec((HID, 2 * HID), lambda: (0, 0)),
                  pl.BlockSpec((8, 2 * HID), lambda: (0, 0)),
                  pl.BlockSpec((2 * HID, HID), lambda: (0, 0)),
                  pl.BlockSpec((8, HID), lambda: (0, 0))],
        out_specs=pl.BlockSpec((8, HID), lambda: (0, 0)),
        out_shape=jax.ShapeDtypeStruct((8, HID), jnp.float32),
    )(vn8, hsum, w1, b1, w2, b2)


def _task_kernel(g_ref, w1_ref, b1_ref, w2_ref, b2_ref, out_ref):
    y = jax.nn.sigmoid(jnp.dot(g_ref[0:1, :], w1_ref[...],
                               preferred_element_type=jnp.float32) + b1_ref[0:1, :])
    y = jnp.dot(y, w2_ref[...], preferred_element_type=jnp.float32) + b2_ref[0:1, :]
    y = jax.nn.sigmoid(y)
    out_ref[...] = jnp.broadcast_to(y, (8, 128))


def _task_block(g, w1, b1, w2, b2):
    return pl.pallas_call(
        _task_kernel,
        in_specs=[pl.BlockSpec((8, GRAPH_FEATS), lambda: (0, 0)),
                  pl.BlockSpec((GRAPH_FEATS, MID), lambda: (0, 0)),
                  pl.BlockSpec((8, MID), lambda: (0, 0)),
                  pl.BlockSpec((MID, 128), lambda: (0, 0)),
                  pl.BlockSpec((8, 128), lambda: (0, 0))],
        out_specs=pl.BlockSpec((8, 128), lambda: (0, 0)),
        out_shape=jax.ShapeDtypeStruct((8, 128), jnp.float32),
    )(g, w1, b1, w2, b2)


def _fold_bn(W, b, g, bb):
    c = g / jnp.sqrt(1.0 + BN_EPS)
    return W * c[None, :], b * c + bb


def _pad8(v):
    return jnp.broadcast_to(v[None, :], (8, v.shape[0]))


def kernel(node_types, edge_index, edge_feats, params):
    gin = params['gin']
    src = edge_index[0]
    dst = edge_index[1]

    eW = jnp.stack([lp['eW'] for lp in gin])
    eb = jnp.stack([_pad8(lp['eb']) for lp in gin])
    e_all = _edge_embeddings(edge_feats, eW, eb)

    h = jnp.zeros((NP, HID), jnp.float32)
    h = h.at[:N_NODES].set(params['node_emb'][node_types])
    vn8 = _pad8(params['vn_emb'][0])

    for l in range(N_LAYERS):
        lp = gin[l]
        w1, b1 = _fold_bn(lp['W1'], lp['b1'], lp['bng1'], lp['bnb1'])
        w2, b2 = _fold_bn(lp['W2'], lp['b2'], lp['bng2'], lp['bnb2'])
        eps8 = jnp.full((8, 128), 1.0 + lp['eps'], jnp.float32)

        hi = h + vn8[0:1, :]
        m = jnp.maximum(hi[src] + e_all[l], 0.0)
        agg = jax.ops.segment_sum(m, dst, num_segments=NP)

        if l < N_LAYERS - 1:
            h, hsum = _node_block(h, agg, vn8, eps8, w1, _pad8(b1),
                                  w2, _pad8(b2), inner_relu=True)
            vp = params['vn_mlp'][l]
            vw1, vb1 = _fold_bn(vp['W1'], vp['b1'], vp['bng1'], vp['bnb1'])
            vw2, vb2 = _fold_bn(vp['W2'], vp['b2'], vp['bng2'], vp['bnb2'])
            vn8 = _vn_block(vn8, hsum, vw1, _pad8(vb1), vw2, _pad8(vb2))
        else:
            g = _final_block(h, agg, vn8, eps8, w1, _pad8(b1), w2, _pad8(b2),
                             params['riW'], _pad8(params['rib']),
                             params['roW'], _pad8(params['rob']))

    t2W = jnp.zeros((MID, 128), jnp.float32).at[:, :TGT].set(params['t2W'])
    t2b = jnp.zeros((128,), jnp.float32).at[:TGT].set(params['t2b'])
    y = _task_block(g, params['t1W'], _pad8(params['t1b']), t2W, _pad8(t2b))
    return y[0:1, :TGT]


# speedup vs baseline: 1.6427x; 1.6427x over previous
import functools

import jax
import jax.numpy as jnp
from jax import lax
from jax.experimental import pallas as pl
from jax.experimental.pallas import tpu as pltpu
from jax.experimental.pallas import tpu_sc as plsc

N_NODES = 10000
N_EDGES = 160000
D_EDGE = 16
HID = 256
GRAPH_FEATS = 512
MID = 256
TGT = 10
N_LAYERS = 5
BN_EPS = 1e-5

NP = 10240
NT = NP // 256
ET = N_EDGES // 256

N_WORKERS = 32
EPW = N_EDGES // N_WORKERS
CK = 40
NCHUNK = EPW // CK


@functools.lru_cache(maxsize=None)
def _get_sc_messages():
    mesh = plsc.VectorSubcoreMesh(core_axis_name="c", subcore_axis_name="s")
    return functools.partial(
        pl.kernel,
        mesh=mesh,
        out_type=jax.ShapeDtypeStruct((N_EDGES, HID), jnp.float32),
        scratch_types=[
            pltpu.VMEM((2, CK), jnp.int32),
            pltpu.VMEM((2, CK, HID), jnp.float32),
            pltpu.VMEM((2, CK, HID), jnp.float32),
            pltpu.VMEM((HID,), jnp.float32),
            pltpu.SemaphoreType.DMA,
            pltpu.SemaphoreType.DMA,
            pltpu.SemaphoreType.DMA,
            pltpu.SemaphoreType.DMA,
            pltpu.SemaphoreType.DMA,
            pltpu.SemaphoreType.DMA,
        ],
    )(_sc_msg_body)


def _sc_msg_body(h_hbm, e_hbm, src_hbm, vn_hbm, m_hbm,
                 src_v, rows_v, e_v, vn_v,
                 sem_g0, sem_g1, sem_e0, sem_e1, sem_w0, sem_w1):
    c = lax.axis_index("c")
    s = lax.axis_index("s")
    wid = s * 2 + c
    base = wid * EPW
    sems_g = (sem_g0, sem_g1)
    sems_e = (sem_e0, sem_e1)
    sems_w = (sem_w0, sem_w1)

    pltpu.sync_copy(vn_hbm, vn_v)
    vn_regs = [vn_v[pl.ds(j * 16, 16)] for j in range(HID // 16)]

    def wait_write(k, b):
        pltpu.make_async_copy(rows_v.at[b],
                              m_hbm.at[pl.ds(base + k * CK, CK)],
                              sems_w[b]).wait()

    def issue(k, b, prev_write=None):
        if prev_write is not None:
            wait_write(prev_write, b)
        o = base + k * CK
        pltpu.sync_copy(src_hbm.at[pl.ds(o, CK)], src_v.at[b])
        pltpu.async_copy(h_hbm.at[src_v.at[b]], rows_v.at[b], sems_g[b])
        pltpu.async_copy(e_hbm.at[pl.ds(o, CK)], e_v.at[b], sems_e[b])

    def process(k, b):
        pltpu.make_async_copy(h_hbm.at[src_v.at[b]], rows_v.at[b],
                              sems_g[b]).wait()
        pltpu.make_async_copy(e_hbm.at[pl.ds(base + k * CK, CK)],
                              e_v.at[b], sems_e[b]).wait()

        def vrow(i, cy):
            for j in range(HID // 16):
                sl = pl.ds(j * 16, 16)
                hin = rows_v[b, i, sl] + vn_regs[j]
                rows_v[b, i, sl] = jnp.maximum(hin + e_v[b, i, sl], 0.0)
            return cy

        lax.fori_loop(0, CK, vrow, 0, unroll=2)
        pltpu.async_copy(rows_v.at[b], m_hbm.at[pl.ds(base + k * CK, CK)],
                         sems_w[b])

    issue(0, 0)
    issue(1, 1)

    def pair(kk, cy):
        k0 = 2 * kk
        process(k0, 0)
        issue(k0 + 2, 0, prev_write=k0)
        process(k0 + 1, 1)

        @pl.when(k0 + 3 < NCHUNK)
        def _():
            issue(k0 + 3, 1, prev_write=k0 + 1)

        return cy

    lax.fori_loop(0, (NCHUNK - 1) // 2, pair, 0)
    process(NCHUNK - 1, 0)
    wait_write(NCHUNK - 1, 0)
    wait_write(NCHUNK - 2, 1)


def _sc_messages(h, e, src, vn_flat):
    return _get_sc_messages()(h, e, src, vn_flat)


def _edge_kernel(ef_ref, w_ref, b_ref, out_ref):
    e = jnp.dot(ef_ref[...], w_ref[...], preferred_element_type=jnp.float32)
    out_ref[...] = e + b_ref[0:1, :]


def _edge_block(edge_feats, eW, eb8):
    return pl.pallas_call(
        _edge_kernel,
        grid=(ET,),
        in_specs=[
            pl.BlockSpec((256, D_EDGE), lambda i: (i, 0)),
            pl.BlockSpec((D_EDGE, HID), lambda i: (0, 0)),
            pl.BlockSpec((8, HID), lambda i: (0, 0)),
        ],
        out_specs=pl.BlockSpec((256, HID), lambda i: (i, 0)),
        out_shape=jax.ShapeDtypeStruct((N_EDGES, HID), jnp.float32),
    )(edge_feats, eW, eb8)


def _node_kernel(h_ref, agg_ref, vn_ref, eps_ref, w1_ref, b1_ref, w2_ref,
                 b2_ref, hout_ref, hsum_ref, *, inner_relu):
    i = pl.program_id(0)
    hi = h_ref[...] + vn_ref[0:1, :]
    s = eps_ref[0, 0]
    z = s * hi + agg_ref[...]
    z = jnp.maximum(jnp.dot(z, w1_ref[...], preferred_element_type=jnp.float32)
                    + b1_ref[0:1, :], 0.0)
    z = jnp.dot(z, w2_ref[...], preferred_element_type=jnp.float32) + b2_ref[0:1, :]
    if inner_relu:
        z = jnp.maximum(z, 0.0)
    ho = z + hi
    row = i * 256 + jax.lax.broadcasted_iota(jnp.int32, (256, HID), 0)
    valid = row < N_NODES
    hout_ref[...] = jnp.where(valid, ho, 0.0)
    him = jnp.where(valid, hi, 0.0)
    colsum = jnp.sum(him, axis=0, keepdims=True)

    @pl.when(i == 0)
    def _():
        hsum_ref[...] = jnp.zeros_like(hsum_ref)

    hsum_ref[0:1, :] = hsum_ref[0:1, :] + colsum


def _node_block(h, agg, vn8, eps8, w1, b1, w2, b2, inner_relu):
    return pl.pallas_call(
        functools.partial(_node_kernel, inner_relu=inner_relu),
        grid=(NT,),
        in_specs=[
            pl.BlockSpec((256, HID), lambda i: (i, 0)),
            pl.BlockSpec((256, HID), lambda i: (i, 0)),
            pl.BlockSpec((8, HID), lambda i: (0, 0)),
            pl.BlockSpec((8, 128), lambda i: (0, 0)),
            pl.BlockSpec((HID, 2 * HID), lambda i: (0, 0)),
            pl.BlockSpec((8, 2 * HID), lambda i: (0, 0)),
            pl.BlockSpec((2 * HID, HID), lambda i: (0, 0)),
            pl.BlockSpec((8, HID), lambda i: (0, 0)),
        ],
        out_specs=[
            pl.BlockSpec((256, HID), lambda i: (i, 0)),
            pl.BlockSpec((8, HID), lambda i: (0, 0)),
        ],
        out_shape=[
            jax.ShapeDtypeStruct((NP, HID), jnp.float32),
            jax.ShapeDtypeStruct((8, HID), jnp.float32),
        ],
    )(h, agg, vn8, eps8, w1, b1, w2, b2)


def _final_kernel(h_ref, agg_ref, vn_ref, eps_ref, w1_ref, b1_ref, w2_ref,
                  b2_ref, riw_ref, rib_ref, row_ref, rob_ref, g_ref):
    i = pl.program_id(0)
    hi = h_ref[...] + vn_ref[0:1, :]
    s = eps_ref[0, 0]
    z = s * hi + agg_ref[...]
    z = jnp.maximum(jnp.dot(z, w1_ref[...], preferred_element_type=jnp.float32)
                    + b1_ref[0:1, :], 0.0)
    z = jnp.dot(z, w2_ref[...], preferred_element_type=jnp.float32) + b2_ref[0:1, :]
    ho = z + hi
    r = jax.nn.sigmoid(jnp.dot(ho, riw_ref[...], preferred_element_type=jnp.float32)
                       + rib_ref[0:1, :])
    r = jnp.dot(r, row_ref[...], preferred_element_type=jnp.float32) + rob_ref[0:1, :]
    rowid = i * 256 + jax.lax.broadcasted_iota(jnp.int32, (256, GRAPH_FEATS), 0)
    r = jnp.where(rowid < N_NODES, r, 0.0)
    colsum = jnp.sum(r, axis=0, keepdims=True)

    @pl.when(i == 0)
    def _():
        g_ref[...] = jnp.zeros_like(g_ref)

    g_ref[0:1, :] = g_ref[0:1, :] + colsum


def _final_block(h, agg, vn8, eps8, w1, b1, w2, b2, riw, rib, row_, rob):
    return pl.pallas_call(
        _final_kernel,
        grid=(NT,),
        in_specs=[
            pl.BlockSpec((256, HID), lambda i: (i, 0)),
            pl.BlockSpec((256, HID), lambda i: (i, 0)),
            pl.BlockSpec((8, HID), lambda i: (0, 0)),
            pl.BlockSpec((8, 128), lambda i: (0, 0)),
            pl.BlockSpec((HID, 2 * HID), lambda i: (0, 0)),
            pl.BlockSpec((8, 2 * HID), lambda i: (0, 0)),
            pl.BlockSpec((2 * HID, HID), lambda i: (0, 0)),
            pl.BlockSpec((8, HID), lambda i: (0, 0)),
            pl.BlockSpec((HID, GRAPH_FEATS), lambda i: (0, 0)),
            pl.BlockSpec((8, GRAPH_FEATS), lambda i: (0, 0)),
            pl.BlockSpec((GRAPH_FEATS, GRAPH_FEATS), lambda i: (0, 0)),
            pl.BlockSpec((8, GRAPH_FEATS), lambda i: (0, 0)),
        ],
        out_specs=pl.BlockSpec((8, GRAPH_FEATS), lambda i: (0, 0)),
        out_shape=jax.ShapeDtypeStruct((8, GRAPH_FEATS), jnp.float32),
    )(h, agg, vn8, eps8, w1, b1, w2, b2, riw, rib, row_, rob)


def _vn_kernel(vn_ref, hs_ref, w1_ref, b1_ref, w2_ref, b2_ref, out_ref):
    vt = hs_ref[0:1, :] + vn_ref[0:1, :]
    a = jnp.maximum(jnp.dot(vt, w1_ref[...], preferred_element_type=jnp.float32)
                    + b1_ref[0:1, :], 0.0)
    b = jnp.dot(a, w2_ref[...], preferred_element_type=jnp.float32) + b2_ref[0:1, :]
    vnn = vn_ref[0:1, :] + jnp.maximum(b, 0.0)
    out_ref[...] = jnp.broadcast_to(vnn, (8, HID))


def _vn_block(vn8, hsum, w1, b1, w2, b2):
    return pl.pallas_call(
        _vn_kernel,
        in_specs=[pl.BlockSpec((8, HID), lambda: (0, 0)),
                  pl.BlockSpec((8, HID), lambda: (0, 0)),
                  pl.BlockSpec((HID, 2 * HID), lambda: (0, 0)),
                  pl.BlockSpec((8, 2 * HID), lambda: (0, 0)),
                  pl.BlockSpec((2 * HID, HID), lambda: (0, 0)),
                  pl.BlockSpec((8, HID), lambda: (0, 0))],
        out_specs=pl.BlockSpec((8, HID), lambda: (0, 0)),
        out_shape=jax.ShapeDtypeStruct((8, HID), jnp.float32),
    )(vn8, hsum, w1, b1, w2, b2)


def _task_kernel(g_ref, w1_ref, b1_ref, w2_ref, b2_ref, out_ref):
    y = jax.nn.sigmoid(jnp.dot(g_ref[0:1, :], w1_ref[...],
                               preferred_element_type=jnp.float32) + b1_ref[0:1, :])
    y = jnp.dot(y, w2_ref[...], preferred_element_type=jnp.float32) + b2_ref[0:1, :]
    y = jax.nn.sigmoid(y)
    out_ref[...] = jnp.broadcast_to(y, (8, 128))


def _task_block(g, w1, b1, w2, b2):
    return pl.pallas_call(
        _task_kernel,
        in_specs=[pl.BlockSpec((8, GRAPH_FEATS), lambda: (0, 0)),
                  pl.BlockSpec((GRAPH_FEATS, MID), lambda: (0, 0)),
                  pl.BlockSpec((8, MID), lambda: (0, 0)),
                  pl.BlockSpec((MID, 128), lambda: (0, 0)),
                  pl.BlockSpec((8, 128), lambda: (0, 0))],
        out_specs=pl.BlockSpec((8, 128), lambda: (0, 0)),
        out_shape=jax.ShapeDtypeStruct((8, 128), jnp.float32),
    )(g, w1, b1, w2, b2)


def _fold_bn(W, b, g, bb):
    c = g / jnp.sqrt(1.0 + BN_EPS)
    return W * c[None, :], b * c + bb


def _pad8(v):
    return jnp.broadcast_to(v[None, :], (8, v.shape[0]))


def kernel(node_types, edge_index, edge_feats, params):
    gin = params['gin']
    src = edge_index[0].astype(jnp.int32)
    dst = edge_index[1]

    h = jnp.zeros((NP, HID), jnp.float32)
    h = h.at[:N_NODES].set(params['node_emb'][node_types])
    vn8 = _pad8(params['vn_emb'][0])

    for l in range(N_LAYERS):
        lp = gin[l]
        w1, b1 = _fold_bn(lp['W1'], lp['b1'], lp['bng1'], lp['bnb1'])
        w2, b2 = _fold_bn(lp['W2'], lp['b2'], lp['bng2'], lp['bnb2'])
        eps8 = jnp.full((8, 128), 1.0 + lp['eps'], jnp.float32)

        e = _edge_block(edge_feats, lp['eW'], _pad8(lp['eb']))
        m = _sc_messages(h, e, src, vn8[0])
        agg = jax.ops.segment_sum(m, dst, num_segments=NP)

        if l < N_LAYERS - 1:
            h, hsum = _node_block(h, agg, vn8, eps8, w1, _pad8(b1),
                                  w2, _pad8(b2), inner_relu=True)
            vp = params['vn_mlp'][l]
            vw1, vb1 = _fold_bn(vp['W1'], vp['b1'], vp['bng1'], vp['bnb1'])
            vw2, vb2 = _fold_bn(vp['W2'], vp['b2'], vp['bng2'], vp['bnb2'])
            vn8 = _vn_block(vn8, hsum, vw1, _pad8(vb1), vw2, _pad8(vb2))
        else:
            g = _final_block(h, agg, vn8, eps8, w1, _pad8(b1), w2, _pad8(b2),
                             params['riW'], _pad8(params['rib']),
                             params['roW'], _pad8(params['rob']))

    t2W = jnp.zeros((MID, 128), jnp.float32).at[:, :TGT].set(params['t2W'])
    t2b = jnp.zeros((128,), jnp.float32).at[:TGT].set(params['t2b'])
    y = _task_block(g, params['t1W'], _pad8(params['t1b']), t2W, _pad8(t2b))
    return y[0:1, :TGT]
